# baseline (speedup 1.0000x reference)
import functools

import jax
import jax.numpy as jnp
from jax import lax
from jax.experimental import pallas as pl
from jax.experimental.pallas import tpu as pltpu
from jax.experimental.pallas import tpu_sc as plsc

NUM_EXPERTS = 8
D_MODEL = 1024
D_FF = 2048
TOKENS = 2048

BM = 256
NT_MAX = 2 * TOKENS // BM + NUM_EXPERTS - 1
S_MAX = NT_MAX * BM

NW = 32
TOK_W = TOKENS // NW
SUB = 32


def _cumsum_tokens(a):
    inc = a
    s = 1
    while s < a.shape[0]:
        shifted = jnp.concatenate([jnp.zeros((s, a.shape[1]), a.dtype), inc[:-s]], axis=0)
        inc = inc + shifted
        s *= 2
    return inc - a


def _router_body(x_ref, wg_ref, combine_ref, pos0_ref, pos1_ref, g0_ref,
                 g1_ref, meta_ref):
    x = x_ref[...]
    wg = wg_ref[...]
    logits = lax.dot_general(
        x, wg, (((1,), (0,)), ((), ())), preferred_element_type=jnp.float32
    )
    iota = lax.broadcasted_iota(jnp.int32, logits.shape, 1)
    big = jnp.int32(NUM_EXPERTS)
    v0 = jnp.max(logits, axis=1, keepdims=True)
    i0 = jnp.min(jnp.where(logits == v0, iota, big), axis=1, keepdims=True)
    masked = jnp.where(iota == i0, -jnp.inf, logits)
    v1 = jnp.max(masked, axis=1, keepdims=True)
    i1 = jnp.min(jnp.where(masked == v1, iota, big), axis=1, keepdims=True)
    e1 = jnp.exp(v1 - v0)
    denom = 1.0 + e1
    g0 = 1.0 / denom
    g1 = e1 / denom
    oh0 = (iota == i0).astype(jnp.int32)
    oh1 = (iota == i1).astype(jnp.int32)
    combine_ref[...] = jnp.where(iota == i0, g0, 0.0) + jnp.where(iota == i1, g1, 0.0)
    g0_ref[...] = jnp.broadcast_to(g0, (TOKENS, 16))
    g1_ref[...] = jnp.broadcast_to(g1, (TOKENS, 16))

    cum_both = _cumsum_tokens(oh0 + oh1)
    rank0 = cum_both
    rank1 = cum_both + oh0
    counts = jnp.sum(oh0 + oh1, axis=0, keepdims=True)
    tiles_e = (counts + BM - 1) // BM
    end_tiles = tiles_e
    s = 1
    while s < NUM_EXPERTS:
        end_tiles = end_tiles + jnp.concatenate(
            [jnp.zeros((1, s), jnp.int32), end_tiles[:, :-s]], axis=1)
        s *= 2
    start_slots = (end_tiles - tiles_e) * BM
    pos0_ref[...] = jnp.sum(oh0 * (start_slots + rank0), axis=1, keepdims=True)
    pos1_ref[...] = jnp.sum(oh1 * (start_slots + rank1), axis=1, keepdims=True)

    n_used = jnp.max(end_tiles)
    lane8 = lax.broadcasted_iota(jnp.int32, (NT_MAX + 1, NUM_EXPERTS), 1)
    tid = lax.broadcasted_iota(jnp.int32, (NT_MAX + 1, NUM_EXPERTS), 0)
    end_b = jnp.broadcast_to(end_tiles, (NT_MAX + 1, NUM_EXPERTS))
    te = jnp.sum((tid >= end_b).astype(jnp.int32), axis=1, keepdims=True)
    last_e = jnp.max(jnp.where(tiles_e > 0, lane8[:1], 0))
    te = jnp.where(te >= NUM_EXPERTS, last_e, te)
    row = lax.broadcasted_iota(jnp.int32, (NT_MAX + 1, 1), 0)
    meta_ref[...] = jnp.where(row == NT_MAX, n_used, te)


def _router(x, Wg):
    return pl.pallas_call(
        _router_body,
        out_shape=(
            jax.ShapeDtypeStruct((TOKENS, NUM_EXPERTS), jnp.float32),
            jax.ShapeDtypeStruct((TOKENS, 1), jnp.int32),
            jax.ShapeDtypeStruct((TOKENS, 1), jnp.int32),
            jax.ShapeDtypeStruct((TOKENS, 16), jnp.float32),
            jax.ShapeDtypeStruct((TOKENS, 16), jnp.float32),
            jax.ShapeDtypeStruct((NT_MAX + 1, 1), jnp.int32),
        ),
    )(x, Wg)


def _dispatch_kernel(x, pos0, pos1):
    mesh = plsc.VectorSubcoreMesh(core_axis_name="c", subcore_axis_name="s")

    @functools.partial(
        pl.kernel,
        out_type=jax.ShapeDtypeStruct((S_MAX, D_MODEL), jnp.float32),
        mesh=mesh,
        scratch_types=[
            pltpu.VMEM((TOK_W, D_MODEL), jnp.float32),
            pltpu.VMEM((TOK_W,), jnp.int32),
            pltpu.VMEM((TOK_W,), jnp.int32),
            pltpu.SemaphoreType.DMA,
        ],
    )
    def body(x_hbm, pos0_hbm, pos1_hbm, xbuf_hbm, rows_v, p0_v, p1_v, sem):
        wid = lax.axis_index("s") * 2 + lax.axis_index("c")
        t0 = wid * TOK_W
        pltpu.sync_copy(pos0_hbm.at[pl.ds(t0, TOK_W)], p0_v)
        pltpu.sync_copy(pos1_hbm.at[pl.ds(t0, TOK_W)], p1_v)
        pltpu.sync_copy(x_hbm.at[pl.ds(t0, TOK_W)], rows_v)
        c0 = pltpu.async_copy(rows_v, xbuf_hbm.at[p0_v], sem)
        c1 = pltpu.async_copy(rows_v, xbuf_hbm.at[p1_v], sem)
        c0.wait()
        c1.wait()

    return body(x, pos0, pos1)


def _groupmm_body(meta_ref, x_ref, w1_ref, b1_ref, w2_ref, b2_ref, out_ref):
    i = pl.program_id(0)
    n_used = meta_ref[NT_MAX]

    @pl.when(i < n_used)
    def _compute():
        x = x_ref[...].astype(jnp.bfloat16)
        h = lax.dot_general(
            x, w1_ref[0], (((1,), (0,)), ((), ())),
            preferred_element_type=jnp.float32) + b1_ref[0]
        h = jnp.maximum(h, 0.0).astype(jnp.bfloat16)
        out_ref[...] = lax.dot_general(
            h, w2_ref[0], (((1,), (0,)), ((), ())),
            preferred_element_type=jnp.float32) + b2_ref[0]


def _groupmm(meta, xbuf, W1, b1, W2, b2):
    grid_spec = pltpu.PrefetchScalarGridSpec(
        num_scalar_prefetch=1,
        grid=(NT_MAX,),
        in_specs=[
            pl.BlockSpec((BM, D_MODEL), lambda i, m: (i, 0)),
            pl.BlockSpec((1, D_MODEL, D_FF), lambda i, m: (m[i], 0, 0)),
            pl.BlockSpec((1, 1, D_FF), lambda i, m: (m[i], 0, 0)),
            pl.BlockSpec((1, D_FF, D_MODEL), lambda i, m: (m[i], 0, 0)),
            pl.BlockSpec((1, 1, D_MODEL), lambda i, m: (m[i], 0, 0)),
        ],
        out_specs=pl.BlockSpec((BM, D_MODEL), lambda i, m: (i, 0)),
    )
    return pl.pallas_call(
        _groupmm_body,
        grid_spec=grid_spec,
        out_shape=jax.ShapeDtypeStruct((S_MAX, D_MODEL), jnp.float32),
    )(meta, xbuf, W1.astype(jnp.bfloat16), b1.reshape(NUM_EXPERTS, 1, D_FF),
      W2.astype(jnp.bfloat16), b2.reshape(NUM_EXPERTS, 1, D_MODEL))


def _combine_kernel(ybuf, pos0, pos1, g0, g1):
    mesh = plsc.VectorSubcoreMesh(core_axis_name="c", subcore_axis_name="s")

    @functools.partial(
        pl.kernel,
        out_type=jax.ShapeDtypeStruct((TOKENS, D_MODEL), jnp.float32),
        mesh=mesh,
        scratch_types=[
            pltpu.VMEM((SUB, D_MODEL), jnp.float32),
            pltpu.VMEM((SUB, D_MODEL), jnp.float32),
            pltpu.VMEM((SUB,), jnp.int32),
            pltpu.VMEM((SUB,), jnp.int32),
            pltpu.VMEM((SUB, 16), jnp.float32),
            pltpu.VMEM((SUB, 16), jnp.float32),
            pltpu.SemaphoreType.DMA,
        ],
    )
    def body(ybuf_hbm, pos0_hbm, pos1_hbm, g0_hbm, g1_hbm, out_hbm,
             r0_v, r1_v, p0_v, p1_v, g0_v, g1_v, sem):
        wid = lax.axis_index("s") * 2 + lax.axis_index("c")

        def sub_chunk(sub, _):
            t0 = wid * TOK_W + sub * SUB
            pltpu.sync_copy(pos0_hbm.at[pl.ds(t0, SUB)], p0_v)
            pltpu.sync_copy(pos1_hbm.at[pl.ds(t0, SUB)], p1_v)
            pltpu.sync_copy(g0_hbm.at[pl.ds(t0, SUB)], g0_v)
            pltpu.sync_copy(g1_hbm.at[pl.ds(t0, SUB)], g1_v)
            c0 = pltpu.async_copy(ybuf_hbm.at[p0_v], r0_v, sem)
            c1 = pltpu.async_copy(ybuf_hbm.at[p1_v], r1_v, sem)
            c0.wait()
            c1.wait()

            def tok(i, _):
                ga = g0_v[i, :]
                gb = g1_v[i, :]
                for d in range(D_MODEL // 16):
                    sl = pl.ds(d * 16, 16)
                    r0_v[i, sl] = r0_v[i, sl] * ga + r1_v[i, sl] * gb
                return 0

            lax.fori_loop(0, SUB, tok, 0)
            pltpu.sync_copy(r0_v, out_hbm.at[pl.ds(t0, SUB)])
            return 0

        lax.fori_loop(0, TOK_W // SUB, sub_chunk, 0, unroll=False)

    return body(ybuf, pos0, pos1, g0, g1)


@jax.jit
def kernel(x, Wg, W1, b1, W2, b2):
    combine, pos0, pos1, g0, g1, meta = _router(x, Wg)
    pos0 = pos0.reshape(TOKENS)
    pos1 = pos1.reshape(TOKENS)
    meta = meta.reshape(NT_MAX + 1)
    xbuf = _dispatch_kernel(x, pos0, pos1)
    ybuf = _groupmm(meta, xbuf, W1, b1, W2, b2)
    out = _combine_kernel(ybuf, pos0, pos1, g0, g1)
    return (out, combine)


# speedup vs baseline: 1.3991x; 1.3991x over previous
import functools

import jax
import jax.numpy as jnp
from jax import lax
from jax.experimental import pallas as pl
from jax.experimental.pallas import tpu as pltpu
from jax.experimental.pallas import tpu_sc as plsc

NUM_EXPERTS = 8
D_MODEL = 1024
D_FF = 2048
TOKENS = 2048

BM = 512
NT_MAX = 2 * TOKENS // BM + NUM_EXPERTS - 1
S_MAX = NT_MAX * BM

NW = 32
TOK_W = TOKENS // NW


def _cumsum_tokens(a):
    inc = a
    s = 1
    while s < a.shape[0]:
        shifted = jnp.concatenate([jnp.zeros((s, a.shape[1]), a.dtype), inc[:-s]], axis=0)
        inc = inc + shifted
        s *= 2
    return inc - a


def _router_body(x_ref, wg_ref, combine_ref, pos0_ref, pos1_ref, g0_ref,
                 g1_ref, meta_ref):
    x = x_ref[...]
    wg = wg_ref[...]
    logits = lax.dot_general(
        x, wg, (((1,), (0,)), ((), ())), preferred_element_type=jnp.float32
    )
    iota = lax.broadcasted_iota(jnp.int32, logits.shape, 1)
    big = jnp.int32(NUM_EXPERTS)
    v0 = jnp.max(logits, axis=1, keepdims=True)
    i0 = jnp.min(jnp.where(logits == v0, iota, big), axis=1, keepdims=True)
    masked = jnp.where(iota == i0, -jnp.inf, logits)
    v1 = jnp.max(masked, axis=1, keepdims=True)
    i1 = jnp.min(jnp.where(masked == v1, iota, big), axis=1, keepdims=True)
    e1 = jnp.exp(v1 - v0)
    denom = 1.0 + e1
    g0 = 1.0 / denom
    g1 = e1 / denom
    oh0 = (iota == i0).astype(jnp.int32)
    oh1 = (iota == i1).astype(jnp.int32)
    combine_ref[...] = jnp.where(iota == i0, g0, 0.0) + jnp.where(iota == i1, g1, 0.0)
    g0_ref[...] = jnp.broadcast_to(g0, (TOKENS, 16))
    g1_ref[...] = jnp.broadcast_to(g1, (TOKENS, 16))

    cum_both = _cumsum_tokens(oh0 + oh1)
    rank0 = cum_both
    rank1 = cum_both + oh0
    counts = jnp.sum(oh0 + oh1, axis=0, keepdims=True)
    tiles_e = (counts + BM - 1) // BM
    end_tiles = tiles_e
    s = 1
    while s < NUM_EXPERTS:
        end_tiles = end_tiles + jnp.concatenate(
            [jnp.zeros((1, s), jnp.int32), end_tiles[:, :-s]], axis=1)
        s *= 2
    start_slots = (end_tiles - tiles_e) * BM
    pos0_ref[...] = jnp.sum(oh0 * (start_slots + rank0), axis=1, keepdims=True)
    pos1_ref[...] = jnp.sum(oh1 * (start_slots + rank1), axis=1, keepdims=True)

    n_used = jnp.max(end_tiles)
    lane8 = lax.broadcasted_iota(jnp.int32, (NT_MAX + 1, NUM_EXPERTS), 1)
    tid = lax.broadcasted_iota(jnp.int32, (NT_MAX + 1, NUM_EXPERTS), 0)
    end_b = jnp.broadcast_to(end_tiles, (NT_MAX + 1, NUM_EXPERTS))
    te = jnp.sum((tid >= end_b).astype(jnp.int32), axis=1, keepdims=True)
    last_e = jnp.max(jnp.where(tiles_e > 0, lane8[:1], 0))
    te = jnp.where(te >= NUM_EXPERTS, last_e, te)
    row = lax.broadcasted_iota(jnp.int32, (NT_MAX + 1, 1), 0)
    meta_ref[...] = jnp.where(row == NT_MAX, n_used, te)


def _router(x, Wg):
    return pl.pallas_call(
        _router_body,
        out_shape=(
            jax.ShapeDtypeStruct((TOKENS, NUM_EXPERTS), jnp.float32),
            jax.ShapeDtypeStruct((TOKENS, 1), jnp.int32),
            jax.ShapeDtypeStruct((TOKENS, 1), jnp.int32),
            jax.ShapeDtypeStruct((TOKENS, 16), jnp.float32),
            jax.ShapeDtypeStruct((TOKENS, 16), jnp.float32),
            jax.ShapeDtypeStruct((NT_MAX + 1, 1), jnp.int32),
        ),
    )(x, Wg)


def _dispatch_kernel(x, pos0, pos1):
    mesh = plsc.VectorSubcoreMesh(core_axis_name="c", subcore_axis_name="s")

    @functools.partial(
        pl.kernel,
        out_type=jax.ShapeDtypeStruct((S_MAX, D_MODEL), jnp.float32),
        mesh=mesh,
        scratch_types=[
            pltpu.VMEM((TOK_W, D_MODEL), jnp.float32),
            pltpu.VMEM((TOK_W,), jnp.int32),
            pltpu.VMEM((TOK_W,), jnp.int32),
            pltpu.SemaphoreType.DMA,
            pltpu.SemaphoreType.DMA,
        ],
    )
    def body(x_hbm, pos0_hbm, pos1_hbm, xbuf_hbm, rows_v, p0_v, p1_v, sem, sem2):
        wid = lax.axis_index("s") * 2 + lax.axis_index("c")
        t0 = wid * TOK_W
        ci = pltpu.async_copy(x_hbm.at[pl.ds(t0, TOK_W)], rows_v, sem2)
        pltpu.sync_copy(pos0_hbm.at[pl.ds(t0, TOK_W)], p0_v)
        pltpu.sync_copy(pos1_hbm.at[pl.ds(t0, TOK_W)], p1_v)
        ci.wait()
        c0 = pltpu.async_copy(rows_v, xbuf_hbm.at[p0_v], sem)
        c1 = pltpu.async_copy(rows_v, xbuf_hbm.at[p1_v], sem)
        c0.wait()
        c1.wait()

    return body(x, pos0, pos1)


def _groupmm_body(meta_ref, x_ref, w1_ref, b1_ref, w2_ref, b2_ref, out_ref):
    i = pl.program_id(0)
    n_used = meta_ref[NT_MAX]

    @pl.when(i < n_used)
    def _compute():
        x = x_ref[...]
        h = lax.dot_general(
            x, w1_ref[0], (((1,), (0,)), ((), ())),
            preferred_element_type=jnp.float32) + b1_ref[0]
        h = jnp.maximum(h, 0.0)
        out_ref[...] = lax.dot_general(
            h, w2_ref[0], (((1,), (0,)), ((), ())),
            preferred_element_type=jnp.float32) + b2_ref[0]


def _groupmm(meta, xbuf, W1, b1, W2, b2):
    grid_spec = pltpu.PrefetchScalarGridSpec(
        num_scalar_prefetch=1,
        grid=(NT_MAX,),
        in_specs=[
            pl.BlockSpec((BM, D_MODEL), lambda i, m: (i, 0)),
            pl.BlockSpec((1, D_MODEL, D_FF), lambda i, m: (m[i], 0, 0)),
            pl.BlockSpec((1, 1, D_FF), lambda i, m: (m[i], 0, 0)),
            pl.BlockSpec((1, D_FF, D_MODEL), lambda i, m: (m[i], 0, 0)),
            pl.BlockSpec((1, 1, D_MODEL), lambda i, m: (m[i], 0, 0)),
        ],
        out_specs=pl.BlockSpec((BM, D_MODEL), lambda i, m: (i, 0)),
    )
    return pl.pallas_call(
        _groupmm_body,
        grid_spec=grid_spec,
        out_shape=jax.ShapeDtypeStruct((S_MAX, D_MODEL), jnp.float32),
    )(meta, xbuf, W1, b1.reshape(NUM_EXPERTS, 1, D_FF), W2,
      b2.reshape(NUM_EXPERTS, 1, D_MODEL))


SUBC = 16
NSUB = TOK_W // SUBC


def _combine_kernel(ybuf, pos0, pos1, g0, g1):
    mesh = plsc.VectorSubcoreMesh(core_axis_name="c", subcore_axis_name="s")

    @functools.partial(
        pl.kernel,
        out_type=jax.ShapeDtypeStruct((TOKENS, D_MODEL), jnp.float32),
        mesh=mesh,
        scratch_types=[
            pltpu.VMEM((SUBC, D_MODEL), jnp.float32),
            pltpu.VMEM((SUBC, D_MODEL), jnp.float32),
            pltpu.VMEM((SUBC, D_MODEL), jnp.float32),
            pltpu.VMEM((SUBC, D_MODEL), jnp.float32),
            pltpu.VMEM((NSUB, SUBC), jnp.int32),
            pltpu.VMEM((NSUB, SUBC), jnp.int32),
            pltpu.VMEM((TOK_W, 16), jnp.float32),
            pltpu.VMEM((TOK_W, 16), jnp.float32),
            pltpu.SemaphoreType.DMA,
            pltpu.SemaphoreType.DMA,
        ],
    )
    def body(ybuf_hbm, pos0_hbm, pos1_hbm, g0_hbm, g1_hbm, out_hbm,
             r0a, r0b, r1a, r1b, p0_v, p1_v, g0_v, g1_v, gsem, osem):
        wid = lax.axis_index("s") * 2 + lax.axis_index("c")
        t0 = wid * TOK_W
        r0 = (r0a, r0b)
        r1 = (r1a, r1b)
        pltpu.sync_copy(pos0_hbm.at[pl.ds(wid * NSUB, NSUB)], p0_v)
        pltpu.sync_copy(pos1_hbm.at[pl.ds(wid * NSUB, NSUB)], p1_v)
        pltpu.sync_copy(g0_hbm.at[pl.ds(t0, TOK_W)], g0_v)
        pltpu.sync_copy(g1_hbm.at[pl.ds(t0, TOK_W)], g1_v)

        def start_gather(s):
            par = s % 2
            c0 = pltpu.async_copy(ybuf_hbm.at[p0_v.at[s]], r0[par], gsem)
            c1 = pltpu.async_copy(ybuf_hbm.at[p1_v.at[s]], r1[par], gsem)
            return (c0, c1)

        copies = {0: start_gather(0)}
        outc = {}
        for s in range(NSUB):
            par = s % 2
            if s + 1 < NSUB:
                if s - 1 >= 0:
                    outc[s - 1].wait()
                copies[s + 1] = start_gather(s + 1)
            copies[s][0].wait()
            copies[s][1].wait()
            r0p, r1p = r0[par], r1[par]

            def tok(i, _, s=s, r0p=r0p, r1p=r1p):
                ga = g0_v[s * SUBC + i, :]
                gb = g1_v[s * SUBC + i, :]
                for d in range(D_MODEL // 16):
                    sl = pl.ds(d * 16, 16)
                    r0p[i, sl] = r0p[i, sl] * ga + r1p[i, sl] * gb
                return 0

            lax.fori_loop(0, SUBC, tok, 0)
            outc[s] = pltpu.async_copy(
                r0p, out_hbm.at[pl.ds(t0 + s * SUBC, SUBC)], osem)
        outc[NSUB - 2].wait()
        outc[NSUB - 1].wait()

    return body(ybuf, pos0, pos1, g0, g1)


@jax.jit
def kernel(x, Wg, W1, b1, W2, b2):
    combine, pos0, pos1, g0, g1, meta = _router(x, Wg)
    pos0 = pos0.reshape(TOKENS)
    pos1 = pos1.reshape(TOKENS)
    meta = meta.reshape(NT_MAX + 1)
    xbuf = _dispatch_kernel(x, pos0, pos1)
    ybuf = _groupmm(meta, xbuf, W1, b1, W2, b2)
    out = _combine_kernel(ybuf, pos0.reshape(NW * NSUB, SUBC),
                          pos1.reshape(NW * NSUB, SUBC), g0, g1)
    return (out, combine)


# speedup vs baseline: 1.4317x; 1.0233x over previous
import functools

import jax
import jax.numpy as jnp
from jax import lax
from jax.experimental import pallas as pl
from jax.experimental.pallas import tpu as pltpu
from jax.experimental.pallas import tpu_sc as plsc

NUM_EXPERTS = 8
D_MODEL = 1024
D_FF = 2048
TOKENS = 2048
DH = D_MODEL // 2

BM = 512
NT_MAX = 2 * TOKENS // BM + NUM_EXPERTS - 1
S_MAX = NT_MAX * BM

NW = 32
TOK_W = TOKENS // NW

_MASK_HI = -65536


def _bf16_round_bits(v):
    b = lax.bitcast_convert_type(v, jnp.int32)
    return b + 0x7FFF + (lax.shift_right_logical(b, 16) & 1)


def _pack_halves(vlo, vhi):
    lo = lax.shift_right_logical(_bf16_round_bits(vlo), 16)
    hi = _bf16_round_bits(vhi) & _MASK_HI
    return lo | hi


def _unpack_halves(w):
    vlo = lax.bitcast_convert_type(lax.shift_left(w, 16), jnp.float32)
    vhi = lax.bitcast_convert_type(w & _MASK_HI, jnp.float32)
    return vlo, vhi


def _cumsum_tokens(a):
    inc = a
    s = 1
    while s < a.shape[0]:
        shifted = jnp.concatenate([jnp.zeros((s, a.shape[1]), a.dtype), inc[:-s]], axis=0)
        inc = inc + shifted
        s *= 2
    return inc - a


def _router_body(x_ref, wg_ref, combine_ref, pos0_ref, pos1_ref, g0_ref,
                 g1_ref, meta_ref, xb_ref):
    x = x_ref[...]
    wg = wg_ref[...]
    logits = lax.dot_general(
        x, wg, (((1,), (0,)), ((), ())), preferred_element_type=jnp.float32
    )
    iota = lax.broadcasted_iota(jnp.int32, logits.shape, 1)
    big = jnp.int32(NUM_EXPERTS)
    v0 = jnp.max(logits, axis=1, keepdims=True)
    i0 = jnp.min(jnp.where(logits == v0, iota, big), axis=1, keepdims=True)
    masked = jnp.where(iota == i0, -jnp.inf, logits)
    v1 = jnp.max(masked, axis=1, keepdims=True)
    i1 = jnp.min(jnp.where(masked == v1, iota, big), axis=1, keepdims=True)
    e1 = jnp.exp(v1 - v0)
    denom = 1.0 + e1
    g0 = 1.0 / denom
    g1 = e1 / denom
    oh0 = (iota == i0).astype(jnp.int32)
    oh1 = (iota == i1).astype(jnp.int32)
    combine_ref[...] = jnp.where(iota == i0, g0, 0.0) + jnp.where(iota == i1, g1, 0.0)
    g0_ref[...] = jnp.broadcast_to(g0, (TOKENS, 16))
    g1_ref[...] = jnp.broadcast_to(g1, (TOKENS, 16))
    xb_ref[...] = _pack_halves(x[:, :DH], x[:, DH:])

    cum_both = _cumsum_tokens(oh0 + oh1)
    rank0 = cum_both
    rank1 = cum_both + oh0
    counts = jnp.sum(oh0 + oh1, axis=0, keepdims=True)
    tiles_e = (counts + BM - 1) // BM
    end_tiles = tiles_e
    s = 1
    while s < NUM_EXPERTS:
        end_tiles = end_tiles + jnp.concatenate(
            [jnp.zeros((1, s), jnp.int32), end_tiles[:, :-s]], axis=1)
        s *= 2
    start_slots = (end_tiles - tiles_e) * BM
    pos0_ref[...] = jnp.sum(oh0 * (start_slots + rank0), axis=1, keepdims=True)
    pos1_ref[...] = jnp.sum(oh1 * (start_slots + rank1), axis=1, keepdims=True)

    n_used = jnp.max(end_tiles)
    lane8 = lax.broadcasted_iota(jnp.int32, (NT_MAX + 1, NUM_EXPERTS), 1)
    tid = lax.broadcasted_iota(jnp.int32, (NT_MAX + 1, NUM_EXPERTS), 0)
    end_b = jnp.broadcast_to(end_tiles, (NT_MAX + 1, NUM_EXPERTS))
    te = jnp.sum((tid >= end_b).astype(jnp.int32), axis=1, keepdims=True)
    last_e = jnp.max(jnp.where(tiles_e > 0, lane8[:1], 0))
    te = jnp.where(te >= NUM_EXPERTS, last_e, te)
    row = lax.broadcasted_iota(jnp.int32, (NT_MAX + 1, 1), 0)
    meta_ref[...] = jnp.where(row == NT_MAX, n_used, te)


def _router(x, Wg):
    return pl.pallas_call(
        _router_body,
        out_shape=(
            jax.ShapeDtypeStruct((TOKENS, NUM_EXPERTS), jnp.float32),
            jax.ShapeDtypeStruct((TOKENS, 1), jnp.int32),
            jax.ShapeDtypeStruct((TOKENS, 1), jnp.int32),
            jax.ShapeDtypeStruct((TOKENS, 16), jnp.float32),
            jax.ShapeDtypeStruct((TOKENS, 16), jnp.float32),
            jax.ShapeDtypeStruct((NT_MAX + 1, 1), jnp.int32),
            jax.ShapeDtypeStruct((TOKENS, DH), jnp.int32),
        ),
    )(x, Wg)


def _dispatch_kernel(xb, pos0, pos1):
    mesh = plsc.VectorSubcoreMesh(core_axis_name="c", subcore_axis_name="s")

    @functools.partial(
        pl.kernel,
        out_type=jax.ShapeDtypeStruct((S_MAX, DH), jnp.int32),
        mesh=mesh,
        scratch_types=[
            pltpu.VMEM((TOK_W, DH), jnp.int32),
            pltpu.VMEM((TOK_W,), jnp.int32),
            pltpu.VMEM((TOK_W,), jnp.int32),
            pltpu.SemaphoreType.DMA,
            pltpu.SemaphoreType.DMA,
        ],
    )
    def body(xb_hbm, pos0_hbm, pos1_hbm, xbuf_hbm, rows_v, p0_v, p1_v, sem, sem2):
        wid = lax.axis_index("s") * 2 + lax.axis_index("c")
        t0 = wid * TOK_W
        ci = pltpu.async_copy(xb_hbm.at[pl.ds(t0, TOK_W)], rows_v, sem2)
        pltpu.sync_copy(pos0_hbm.at[pl.ds(t0, TOK_W)], p0_v)
        pltpu.sync_copy(pos1_hbm.at[pl.ds(t0, TOK_W)], p1_v)
        ci.wait()
        c0 = pltpu.async_copy(rows_v, xbuf_hbm.at[p0_v], sem)
        c1 = pltpu.async_copy(rows_v, xbuf_hbm.at[p1_v], sem)
        c0.wait()
        c1.wait()

    return body(xb, pos0, pos1)


def _groupmm_body(meta_ref, x_ref, w1_ref, b1_ref, w2_ref, b2_ref, out_ref):
    i = pl.program_id(0)
    n_used = meta_ref[NT_MAX]

    @pl.when(i < n_used)
    def _compute():
        xlo, xhi = _unpack_halves(x_ref[...])
        x = jnp.concatenate([xlo, xhi], axis=1)
        h = lax.dot_general(
            x, w1_ref[0], (((1,), (0,)), ((), ())),
            preferred_element_type=jnp.float32) + b1_ref[0]
        h = jnp.maximum(h, 0.0)
        y = lax.dot_general(
            h, w2_ref[0], (((1,), (0,)), ((), ())),
            preferred_element_type=jnp.float32) + b2_ref[0]
        out_ref[...] = _pack_halves(y[:, :DH], y[:, DH:])


def _groupmm(meta, xbuf, W1, b1, W2, b2):
    grid_spec = pltpu.PrefetchScalarGridSpec(
        num_scalar_prefetch=1,
        grid=(NT_MAX,),
        in_specs=[
            pl.BlockSpec((BM, DH),
                         lambda i, m: (jnp.minimum(i, m[NT_MAX] - 1), 0)),
            pl.BlockSpec((1, D_MODEL, D_FF), lambda i, m: (m[i], 0, 0)),
            pl.BlockSpec((1, 1, D_FF), lambda i, m: (m[i], 0, 0)),
            pl.BlockSpec((1, D_FF, D_MODEL), lambda i, m: (m[i], 0, 0)),
            pl.BlockSpec((1, 1, D_MODEL), lambda i, m: (m[i], 0, 0)),
        ],
        out_specs=pl.BlockSpec((BM, DH),
                               lambda i, m: (jnp.minimum(i, m[NT_MAX] - 1), 0)),
    )
    return pl.pallas_call(
        _groupmm_body,
        grid_spec=grid_spec,
        out_shape=jax.ShapeDtypeStruct((S_MAX, DH), jnp.int32),
    )(meta, xbuf, W1, b1.reshape(NUM_EXPERTS, 1, D_FF), W2,
      b2.reshape(NUM_EXPERTS, 1, D_MODEL))


SUBC = 16
NSUB = TOK_W // SUBC


def _combine_kernel(ybuf, pos0, pos1, g0, g1):
    mesh = plsc.VectorSubcoreMesh(core_axis_name="c", subcore_axis_name="s")

    @functools.partial(
        pl.kernel,
        out_type=jax.ShapeDtypeStruct((TOKENS, D_MODEL), jnp.float32),
        mesh=mesh,
        scratch_types=[
            pltpu.VMEM((SUBC, DH), jnp.int32),
            pltpu.VMEM((SUBC, DH), jnp.int32),
            pltpu.VMEM((SUBC, DH), jnp.int32),
            pltpu.VMEM((SUBC, DH), jnp.int32),
            pltpu.VMEM((SUBC, D_MODEL), jnp.float32),
            pltpu.VMEM((SUBC, D_MODEL), jnp.float32),
            pltpu.VMEM((NSUB, SUBC), jnp.int32),
            pltpu.VMEM((NSUB, SUBC), jnp.int32),
            pltpu.VMEM((TOK_W, 16), jnp.float32),
            pltpu.VMEM((TOK_W, 16), jnp.float32),
            pltpu.SemaphoreType.DMA,
            pltpu.SemaphoreType.DMA,
        ],
    )
    def body(ybuf_hbm, pos0_hbm, pos1_hbm, g0_hbm, g1_hbm, out_hbm,
             r0a, r0b, r1a, r1b, oa, ob, p0_v, p1_v, g0_v, g1_v, gsem, osem):
        wid = lax.axis_index("s") * 2 + lax.axis_index("c")
        t0 = wid * TOK_W
        r0 = (r0a, r0b)
        r1 = (r1a, r1b)
        ov = (oa, ob)
        pltpu.sync_copy(pos0_hbm.at[pl.ds(wid * NSUB, NSUB)], p0_v)
        pltpu.sync_copy(pos1_hbm.at[pl.ds(wid * NSUB, NSUB)], p1_v)
        pltpu.sync_copy(g0_hbm.at[pl.ds(t0, TOK_W)], g0_v)
        pltpu.sync_copy(g1_hbm.at[pl.ds(t0, TOK_W)], g1_v)

        def start_gather(s):
            par = s % 2
            c0 = pltpu.async_copy(ybuf_hbm.at[p0_v.at[s]], r0[par], gsem)
            c1 = pltpu.async_copy(ybuf_hbm.at[p1_v.at[s]], r1[par], gsem)
            return (c0, c1)

        copies = {0: start_gather(0)}
        outc = {}
        for s in range(NSUB):
            par = s % 2
            if s + 1 < NSUB:
                if s - 1 >= 0:
                    outc[s - 1].wait()
                copies[s + 1] = start_gather(s + 1)
            copies[s][0].wait()
            copies[s][1].wait()
            r0p, r1p, op = r0[par], r1[par], ov[par]

            def tok(i, _, s=s, r0p=r0p, r1p=r1p, op=op):
                ga = g0_v[s * SUBC + i, :]
                gb = g1_v[s * SUBC + i, :]
                for d in range(DH // 16):
                    sl = pl.ds(d * 16, 16)
                    sh = pl.ds(DH + d * 16, 16)
                    w0 = r0p[i, sl]
                    w1 = r1p[i, sl]
                    a_lo, a_hi = _unpack_halves(w0)
                    b_lo, b_hi = _unpack_halves(w1)
                    op[i, sl] = a_lo * ga + b_lo * gb
                    op[i, sh] = a_hi * ga + b_hi * gb
                return 0

            lax.fori_loop(0, SUBC, tok, 0)
            outc[s] = pltpu.async_copy(
                op, out_hbm.at[pl.ds(t0 + s * SUBC, SUBC)], osem)
        outc[NSUB - 2].wait()
        outc[NSUB - 1].wait()

    return body(ybuf, pos0, pos1, g0, g1)


@jax.jit
def kernel(x, Wg, W1, b1, W2, b2):
    combine, pos0, pos1, g0, g1, meta, xb = _router(x, Wg)
    pos0 = pos0.reshape(TOKENS)
    pos1 = pos1.reshape(TOKENS)
    meta = meta.reshape(NT_MAX + 1)
    xbuf = _dispatch_kernel(xb, pos0, pos1)
    ybuf = _groupmm(meta, xbuf, W1, b1, W2, b2)
    out = _combine_kernel(ybuf, pos0.reshape(NW * NSUB, SUBC),
                          pos1.reshape(NW * NSUB, SUBC), g0, g1)
    return (out, combine)
